# Initial kernel scaffold; baseline (speedup 1.0000x reference)
import jax
import jax.numpy as jnp
from jax.experimental import pallas as pl


def kernel(x, lang_ids, W, b):
    raise NotImplementedError("write your pallas kernel here")


# speedup vs baseline: 1.1953x; 1.1953x over previous
import jax
import jax.numpy as jnp
from jax.experimental import pallas as pl
from jax.experimental.pallas import tpu as pltpu

DICT_LEN = 9
NUM_LS = 8
DIM = 1024
SEQ = 2048
BZ = 16
BLK_S = 128


def _mapper_kernel(idx_ref, x_ref, w_ref, b_ref, o_ref):
    for j in range(BZ):
        e = idx_ref[j]
        xj = x_ref[:, j, :].astype(jnp.bfloat16)
        wj = w_ref[e]
        yj = jax.lax.dot_general(
            xj, wj,
            dimension_numbers=(((1,), (0,)), ((), ())),
            preferred_element_type=jnp.float32,
        )
        o_ref[:, j, :] = yj + b_ref[e]


def kernel(x, lang_ids, W, b):
    idx = jnp.clip(DICT_LEN - 2 - lang_ids, 0, NUM_LS - 1).astype(jnp.int32)
    Wt = jnp.swapaxes(W, 1, 2).astype(jnp.bfloat16)
    grid = (SEQ // BLK_S,)
    out = pl.pallas_call(
        _mapper_kernel,
        grid_spec=pltpu.PrefetchScalarGridSpec(
            num_scalar_prefetch=1,
            grid=grid,
            in_specs=[
                pl.BlockSpec((BLK_S, BZ, DIM), lambda s, idx_ref: (s, 0, 0)),
                pl.BlockSpec((NUM_LS, DIM, DIM), lambda s, idx_ref: (0, 0, 0)),
                pl.BlockSpec((NUM_LS, DIM), lambda s, idx_ref: (0, 0)),
            ],
            out_specs=pl.BlockSpec((BLK_S, BZ, DIM), lambda s, idx_ref: (s, 0, 0)),
        ),
        out_shape=jax.ShapeDtypeStruct((SEQ, BZ, DIM), jnp.float32),
    )(idx, x, Wt, b)
    return out
